# Initial kernel scaffold; baseline (speedup 1.0000x reference)
import jax
import jax.numpy as jnp
from jax.experimental import pallas as pl


def kernel(input_features, doc_ids, loss_mask, ln_g, ln_b, W_q, W_k, W_v, W_g, W_res, xl_u, xl_v, codebook):
    raise NotImplementedError("write your pallas kernel here")


# speedup vs baseline: 2.3490x; 2.3490x over previous
import functools

import jax
import jax.numpy as jnp
from jax import lax
from jax.experimental import pallas as pl
from jax.experimental.pallas import tpu as pltpu
from jax.experimental.pallas import tpu_sc as plsc

MASK_INFTY_APPROX = 1e30

_MM_BF16 = True


def _mm(a, b, dims):
    if _MM_BF16:
        a = a.astype(jnp.bfloat16)
        b = b.astype(jnp.bfloat16)
    return lax.dot_general(a, b, (dims, ((), ())),
                           preferred_element_type=jnp.float32)


def _prep_body(x_ref, g_ref, b_ref, wq_ref, wk_ref, wv_ref, wg_ref,
               xu_ref, xv_ref, ph_ref,
               qe_ref, k_ref, v_ref, gate_ref, *, H, DK, DV):
    x = x_ref[...]
    mu = jnp.mean(x, axis=1, keepdims=True)
    xc = x - mu
    var = jnp.mean(xc * xc, axis=1, keepdims=True)
    xt = xc * lax.rsqrt(var + 1e-6) * g_ref[...] + b_ref[...]
    tau = DK ** 0.5
    q = _mm(xt, wq_ref[...], ((1,), (0,)))
    k = _mm(xt, wk_ref[...], ((1,), (0,)))
    v = _mm(xt, wv_ref[...], ((1,), (0,)))
    g = _mm(xt, wg_ref[...], ((1,), (0,)))
    gate = g * (1.0 / (1.0 + jnp.exp(-g)))
    ph = ph_ref[...]
    F = DK // 2
    sin_l = ph[:, :F]
    cos_l = ph[:, F:]
    for h in range(H):
        qh = q[:, h * DK:(h + 1) * DK]
        m = jnp.mean(qh, axis=1, keepdims=True)
        qhc = qh - m
        vv = jnp.mean(qhc * qhc, axis=1, keepdims=True)
        qh = qhc * lax.rsqrt(vv + 1e-6) * (1.0 / tau)
        kh = k[:, h * DK:(h + 1) * DK]
        m = jnp.mean(kh, axis=1, keepdims=True)
        khc = kh - m
        vv = jnp.mean(khc * khc, axis=1, keepdims=True)
        kh = khc * lax.rsqrt(vv + 1e-6)
        qu = qh + xu_ref[h][None, :]
        qv = qh + xv_ref[h][None, :]
        qs = qv[:, :F]
        qc = qv[:, F:]
        A = qs * sin_l + qc * cos_l
        B = qc * sin_l - qs * cos_l
        qe_ref[h] = jnp.concatenate([qu, A, B], axis=1)
        k_ref[h] = kh
        v_ref[h] = v[:, h * DV:(h + 1) * DV]
        gate_ref[h] = gate[:, h * DV:(h + 1) * DV]


def _vq_body(k_ref, c_ref, mask_ref, maskT_ref,
             z_ref, mv_ref, loss_ref, *, S, BS, NSB):
    h = pl.program_id(0)
    sb = pl.program_id(1)
    k = k_ref[0]
    c = c_ref[0]
    dots = _mm(c, k, ((1,), (1,)))
    csq = jnp.sum(c * c, axis=1, keepdims=True)
    dist = csq - 2.0 * dots
    bmin = jnp.min(dist, axis=0, keepdims=True)
    row = lax.broadcasted_iota(jnp.int32, dist.shape, 0)
    barg = jnp.min(jnp.where(dist == bmin, row, S), axis=0, keepdims=True)
    barg = barg + (sb * BS + h * S)

    @pl.when(sb == 0)
    def _():
        mv_ref[0] = bmin
        z_ref[0] = barg

    @pl.when(sb > 0)
    def _():
        cur = mv_ref[0]
        upd = bmin < cur
        mv_ref[0] = jnp.where(upd, bmin, cur)
        z_ref[0] = jnp.where(upd, barg, z_ref[0])

    @pl.when(sb == NSB - 1)
    def _():
        ksq_mask = jnp.sum((k * k) * maskT_ref[...])
        total = ksq_mask + jnp.sum(mask_ref[...] * mv_ref[0])
        loss_ref[0] = jnp.broadcast_to(total, loss_ref.shape[1:])


def _attn_body(x_ref, qe_ref, kh_ref, p_ref, v_ref, gate_ref, wres_ref,
               out_ref, *, DK, NLB, BL):
    h = pl.program_id(1)
    lb = pl.program_id(0)
    qe = qe_ref[0]
    qu = qe[:, :DK]
    qt = qe[:, DK:]
    s = _mm(qu, kh_ref[0], ((1,), (1,)))
    s += _mm(qt, p_ref[...], ((1,), (1,)))
    rows = lb * BL + lax.broadcasted_iota(jnp.int32, s.shape, 0)
    cols = lax.broadcasted_iota(jnp.int32, s.shape, 1)
    s = jnp.where(cols <= rows, s, -MASK_INFTY_APPROX)
    m = jnp.max(s, axis=1, keepdims=True)
    e = jnp.exp(s - m)
    a = e / jnp.sum(e, axis=1, keepdims=True)
    wv = _mm(a, v_ref[0], ((1,), (0,)))
    o = wv * gate_ref[0]
    contrib = _mm(o, wres_ref[0], ((1,), (0,)))

    @pl.when(h == 0)
    def _():
        out_ref[...] = x_ref[...] + contrib

    @pl.when(h > 0)
    def _():
        out_ref[...] += contrib


def _sc_gather(table, idx, BTOT, D):
    info = plsc.get_sparse_core_info()
    NW = info.num_cores * info.num_subcores
    b_per_w = BTOT // NW
    mesh = plsc.VectorSubcoreMesh(core_axis_name="c", subcore_axis_name="s")

    @functools.partial(
        pl.kernel, mesh=mesh,
        out_type=jax.ShapeDtypeStruct((BTOT, D), jnp.float32),
        compiler_params=pltpu.CompilerParams(use_tc_tiling_on_sc=False),
        scratch_types=[
            pltpu.VMEM((b_per_w,), jnp.int32),
            pltpu.VMEM((b_per_w, D), jnp.float32),
            pltpu.SemaphoreType.DMA,
        ],
    )
    def gk(table_hbm, idx_hbm, out_hbm, idx_v, rows_v, sem):
        wid = lax.axis_index("s") * info.num_cores + lax.axis_index("c")
        base = wid * b_per_w
        pltpu.sync_copy(idx_hbm.at[pl.ds(base, b_per_w)], idx_v)
        pltpu.async_copy(table_hbm.at[idx_v], rows_v, sem).wait()
        pltpu.sync_copy(rows_v, out_hbm.at[pl.ds(base, b_per_w)])

    return gk(table, idx)


def kernel(input_features, doc_ids, loss_mask, ln_g, ln_b, W_q, W_k, W_v,
           W_g, W_res, xl_u, xl_v, codebook):
    B, L, D = input_features.shape
    H, S, DK = codebook.shape
    DV = W_v.shape[1] // H
    x = input_features[0]

    F = DK // 2
    inv = 1.0 / (10000.0 ** (jnp.arange(0, DK, 2).astype(jnp.float32) / DK))
    ang = jnp.arange(L, dtype=jnp.float32)[:, None] * inv[None, :]
    sin_l = jnp.sin(ang)
    cos_l = jnp.cos(ang)
    ph = jnp.concatenate([sin_l, cos_l], axis=1)
    P = jnp.concatenate([cos_l, sin_l], axis=1)

    BL = 256
    NLB = L // BL

    qe, k, v, gate = pl.pallas_call(
        functools.partial(_prep_body, H=H, DK=DK, DV=DV),
        grid=(NLB,),
        in_specs=[
            pl.BlockSpec((BL, D), lambda i: (i, 0)),
            pl.BlockSpec((1, D), lambda i: (0, 0)),
            pl.BlockSpec((1, D), lambda i: (0, 0)),
            pl.BlockSpec((D, H * DK), lambda i: (0, 0)),
            pl.BlockSpec((D, H * DK), lambda i: (0, 0)),
            pl.BlockSpec((D, H * DV), lambda i: (0, 0)),
            pl.BlockSpec((D, H * DV), lambda i: (0, 0)),
            pl.BlockSpec((H, DK), lambda i: (0, 0)),
            pl.BlockSpec((H, DK), lambda i: (0, 0)),
            pl.BlockSpec((BL, DK), lambda i: (i, 0)),
        ],
        out_specs=[
            pl.BlockSpec((H, BL, 2 * DK), lambda i: (0, i, 0)),
            pl.BlockSpec((H, BL, DK), lambda i: (0, i, 0)),
            pl.BlockSpec((H, BL, DV), lambda i: (0, i, 0)),
            pl.BlockSpec((H, BL, DV), lambda i: (0, i, 0)),
        ],
        out_shape=[
            jax.ShapeDtypeStruct((H, L, 2 * DK), jnp.float32),
            jax.ShapeDtypeStruct((H, L, DK), jnp.float32),
            jax.ShapeDtypeStruct((H, L, DV), jnp.float32),
            jax.ShapeDtypeStruct((H, L, DV), jnp.float32),
        ],
    )(x, ln_g[None, :], ln_b[None, :], W_q, W_k, W_v, W_g, xl_u, xl_v, ph)

    BS = 512
    NSB = S // BS
    mask = loss_mask[0][None, :]
    maskT = loss_mask[0][:, None]
    z, mv, lossp = pl.pallas_call(
        functools.partial(_vq_body, S=S, BS=BS, NSB=NSB),
        grid=(H, NSB),
        in_specs=[
            pl.BlockSpec((1, L, DK), lambda h, s: (h, 0, 0)),
            pl.BlockSpec((1, BS, DK), lambda h, s: (h, s, 0)),
            pl.BlockSpec((1, L), lambda h, s: (0, 0)),
            pl.BlockSpec((L, 1), lambda h, s: (0, 0)),
        ],
        out_specs=[
            pl.BlockSpec((1, 1, L), lambda h, s: (h, 0, 0)),
            pl.BlockSpec((1, 1, L), lambda h, s: (h, 0, 0)),
            pl.BlockSpec((1, 1, 128), lambda h, s: (h, 0, 0)),
        ],
        out_shape=[
            jax.ShapeDtypeStruct((H, 1, L), jnp.int32),
            jax.ShapeDtypeStruct((H, 1, L), jnp.float32),
            jax.ShapeDtypeStruct((H, 1, 128), jnp.float32),
        ],
    )(k, codebook, mask, maskT)

    loss = jnp.sum(lossp[:, 0, 0]) / (B * H * L)

    k_hat = _sc_gather(codebook.reshape(H * S, DK), z.reshape(H * L),
                       H * L, DK)
    k_hat = k_hat.reshape(H, L, DK)

    out = pl.pallas_call(
        functools.partial(_attn_body, DK=DK, NLB=NLB, BL=BL),
        grid=(NLB, H),
        in_specs=[
            pl.BlockSpec((BL, D), lambda i, h: (i, 0)),
            pl.BlockSpec((1, BL, 2 * DK), lambda i, h: (h, i, 0)),
            pl.BlockSpec((1, L, DK), lambda i, h: (h, 0, 0)),
            pl.BlockSpec((L, DK), lambda i, h: (0, 0)),
            pl.BlockSpec((1, L, DV), lambda i, h: (h, 0, 0)),
            pl.BlockSpec((1, BL, DV), lambda i, h: (h, i, 0)),
            pl.BlockSpec((1, DV, D), lambda i, h: (h, 0, 0)),
        ],
        out_specs=pl.BlockSpec((BL, D), lambda i, h: (i, 0)),
        out_shape=jax.ShapeDtypeStruct((L, D), jnp.float32),
    )(x, qe, k_hat, P, v, gate, W_res.reshape(H, DV, D))

    return out[None], loss, loss
